# baseline (speedup 1.0000x reference)
import functools

import jax
import jax.numpy as jnp
from jax import lax
from jax.experimental import pallas as pl
from jax.experimental.pallas import tpu as pltpu
from jax.experimental.pallas import tpu_sc as plsc

_V = 100000
_E = 64
_B = 1024
_CTX = 20

_NC = 2
_NS = 16
_NW = _NC * _NS
_ROWS_W = _B * _CTX // _NW
_CHUNK = 128
_NCHUNK = _ROWS_W // _CHUNK
_B_W = _B // _NW


def _sc_gather_sum(idx3, table):
    mesh = plsc.VectorSubcoreMesh(core_axis_name="c", subcore_axis_name="s")

    @functools.partial(
        pl.kernel,
        mesh=mesh,
        compiler_params=pltpu.CompilerParams(use_tc_tiling_on_sc=False),
        out_type=jax.ShapeDtypeStruct((_B, _E), jnp.float32),
        scratch_types=[
            pltpu.VMEM((_NCHUNK, _CHUNK), jnp.int32),
            pltpu.VMEM((_ROWS_W, _E), jnp.float32),
            pltpu.VMEM((_B_W, _E), jnp.float32),
            pltpu.SemaphoreType.DMA,
        ],
    )
    def k(idx_hbm, table_hbm, out_hbm, idx_v, rows_v, acc_v, sem):
        wid = lax.axis_index("s") * _NC + lax.axis_index("c")
        pltpu.sync_copy(idx_hbm.at[wid], idx_v)
        copies = [
            pltpu.async_copy(
                table_hbm.at[idx_v.at[c]],
                rows_v.at[pl.ds(c * _CHUNK, _CHUNK)],
                sem,
            )
            for c in range(_NCHUNK)
        ]
        for cp in copies:
            cp.wait()

        def body(bi, carry):
            base = bi * _CTX
            for c in range(_E // 16):
                acc = rows_v[base, pl.ds(c * 16, 16)]
                for j in range(1, _CTX):
                    acc = acc + rows_v[base + j, pl.ds(c * 16, 16)]
                acc_v[bi, pl.ds(c * 16, 16)] = acc
            return carry

        lax.fori_loop(0, _B_W, body, 0)
        pltpu.sync_copy(acc_v, out_hbm.at[pl.ds(wid * _B_W, _B_W)])

    return k(idx3, table)


_BT = 32
_TV = 1024
_NT = _V // _TV
_TAIL = _V - _NT * _TV


def _tc_body(emb_ref, wt_ref, b_ref, out_ref):
    x = emb_ref[...].astype(jnp.bfloat16)
    m = jnp.full((_BT, 1), -1e30, jnp.float32)
    s = jnp.zeros((_BT, 1), jnp.float32)
    spans = [(t * _TV, _TV) for t in range(_NT)]
    if _TAIL:
        spans.append((_NT * _TV, _TAIL))
    for off, w in spans:
        wt = wt_ref[:, off:off + w]
        logits = lax.dot_general(
            x, wt, (((1,), (0,)), ((), ())),
            preferred_element_type=jnp.float32,
        )
        logits = logits + b_ref[0:1, off:off + w]
        out_ref[:, off:off + w] = logits
        mt = jnp.max(logits, axis=1, keepdims=True)
        mn = jnp.maximum(m, mt)
        s = s * jnp.exp(m - mn) + jnp.sum(jnp.exp(logits - mn), axis=1,
                                          keepdims=True)
        m = mn
    lse = m + jnp.log(s)
    for off, w in spans:
        out_ref[:, off:off + w] = out_ref[:, off:off + w] - lse


def _tc_logsoftmax(emb, wt, b2):
    return pl.pallas_call(
        _tc_body,
        grid=(_B // _BT,),
        in_specs=[
            pl.BlockSpec((_BT, _E), lambda i: (i, 0)),
            pl.BlockSpec((_E, _V), lambda i: (0, 0)),
            pl.BlockSpec((1, _V), lambda i: (0, 0)),
        ],
        out_specs=pl.BlockSpec((_BT, _V), lambda i: (i, 0)),
        out_shape=jax.ShapeDtypeStruct((_B, _V), jnp.float32),
    )(emb, wt, b2)


def kernel(input_word, table, W, b):
    idx3 = input_word.astype(jnp.int32).reshape(_NW, _NCHUNK, _CHUNK)
    emb = _sc_gather_sum(idx3, table)
    wt = W.astype(jnp.bfloat16).T
    b2 = b.reshape(1, _V)
    return _tc_logsoftmax(emb, wt, b2)
